# baseline (speedup 1.0000x reference)
import jax
import jax.numpy as jnp
from jax.experimental import pallas as pl
import jax.experimental.pallas.tpu as pltpu


def _fold128(t):
    acc = t[:, 0:128]
    for q in range(1, t.shape[1] // 128):
        acc = acc + t[:, q * 128:(q + 1) * 128]
    return acc


def _fold8(t):
    acc = t[0:8, :]
    for q in range(1, t.shape[0] // 8):
        acc = acc + t[q * 8:(q + 1) * 8, :]
    return acc


def _prep_kernel(x_ref, s_ref, rsum_ref, acc_ref):
    h = pl.program_id(1)
    nh = pl.num_programs(1)
    s = jnp.exp(x_ref[...] * 50.0)
    s_ref[...] = s
    part = _fold128(s)

    @pl.when(h == 0)
    def _():
        acc_ref[...] = part

    @pl.when(h != 0)
    def _():
        acc_ref[...] += part

    @pl.when(h == nh - 1)
    def _():
        rsum_ref[...] = jnp.sum(acc_ref[...], axis=1, keepdims=True).T


def _fold128_max(o):
    acc = o[:, 0:128]
    for q in range(1, o.shape[1] // 128):
        acc = jnp.maximum(acc, o[:, q * 128:(q + 1) * 128])
    return acc


def _make_mega_kernel(k_res, pbh, cw, nv_res, nh):
    def _mega_kernel(simst_ref, sbot_ref, bsum_ref, out_ref, idx_ref,
                     stop_ref, r_ref, racc_ref, acc8_ref, c_ref, bi_ref):
        p = pl.program_id(0)
        t = pl.program_id(1)
        npass = pl.num_programs(0)
        nt = pl.num_programs(1)
        v = t // nh
        h = t % nh
        res_steps = nv_res * nh
        rows_res = pl.ds(v * pbh, pbh)
        rows_str = pl.ds(k_res + (v - nv_res) * pbh, pbh)
        lanes = pl.ds(h * cw, cw)

        def row_acc(part, rows):
            @pl.when(h == 0)
            def _():
                racc_ref[rows] = part

            @pl.when(h != 0)
            def _():
                racc_ref[rows] += part

            @pl.when(h == nh - 1)
            def _():
                r_ref[rows] = 1.0 / jnp.sum(racc_ref[rows], axis=1,
                                            keepdims=True)

        @pl.when((p == 0) & (t < res_steps))
        def _():
            st = jnp.exp(simst_ref[...] * 50.0)
            stop_ref[t] = st
            row_acc(_fold128(st), rows_res)

        @pl.when((p == 0) & (t == res_steps))
        def _():
            r_ref[k_res:] = (1.0 / bsum_ref[...]).T

        def col_body(src, rows):
            part8 = _fold8(src * r_ref[rows])

            @pl.when(v == 0)
            def _():
                acc8_ref[:, lanes] = part8

            @pl.when(v != 0)
            def _():
                acc8_ref[:, lanes] += part8

            @pl.when(t == nt - 1)
            def _():
                c_ref[...] = 1.0 / jnp.sum(acc8_ref[...], axis=0,
                                           keepdims=True)

        is_col = (p % 2 == 1) & (p < npass - 1)

        @pl.when(is_col & (v < nv_res))
        def _():
            col_body(stop_ref[t], rows_res)

        @pl.when(is_col & (v >= nv_res))
        def _():
            col_body(sbot_ref[...], rows_str)

        is_row = (p % 2 == 0) & (p > 0) & (p < npass - 1)

        @pl.when(is_row & (v < nv_res))
        def _():
            row_acc(_fold128(stop_ref[t] * c_ref[:, lanes]), rows_res)

        @pl.when(is_row & (v >= nv_res))
        def _():
            row_acc(_fold128(sbot_ref[...] * c_ref[:, lanes]), rows_str)

        def out_body(src, rows):
            o = src * r_ref[rows] * c_ref[:, lanes]
            out_ref[...] = o
            bm = jnp.max(_fold128_max(o), axis=1, keepdims=True)
            gi = (jnp.argmax(o, axis=1).reshape(pbh, 1).astype(jnp.int32)
                  + h * cw)

            @pl.when(h == 0)
            def _():
                racc_ref[rows, 0:1] = bm
                bi_ref[rows] = gi

            @pl.when(h != 0)
            def _():
                upd = bm > racc_ref[rows, 0:1]
                racc_ref[rows, 0:1] = jnp.where(upd, bm,
                                                racc_ref[rows, 0:1])
                bi_ref[rows] = jnp.where(upd, gi, bi_ref[rows])

            @pl.when(t == nt - 1)
            def _():
                idx_ref[...] = bi_ref[...].T

        is_out = p == npass - 1

        @pl.when(is_out & (v < nv_res))
        def _():
            out_body(stop_ref[t], rows_res)

        @pl.when(is_out & (v >= nv_res))
        def _():
            out_body(sbot_ref[...], rows_str)

    return _mega_kernel


def kernel(sims, batch_size=256):
    del batch_size
    num_row, num_col = sims.shape
    work = sims.T if num_row >= num_col else sims
    m, n = work.shape

    pbh = min(256, m)
    cw = min(2048, n)
    nh = n // cw
    nv = m // pbh
    nv_res = max(1, (nv * 5) // 16)
    if nv - nv_res < 1:
        nv_res = nv - 1
    k_res = nv_res * pbh
    mb = m - k_res
    npass = 11

    s_bot, bsum = pl.pallas_call(
        _prep_kernel,
        grid=(mb // pbh, nh),
        in_specs=[pl.BlockSpec((pbh, cw),
                               lambda i, j, K=nv_res: (i + K, j))],
        out_specs=[
            pl.BlockSpec((pbh, cw), lambda i, j: (i, j)),
            pl.BlockSpec((1, pbh), lambda i, j: (0, i)),
        ],
        out_shape=[
            jax.ShapeDtypeStruct((mb, n), jnp.float32),
            jax.ShapeDtypeStruct((1, mb), jnp.float32),
        ],
        scratch_shapes=[pltpu.VMEM((pbh, 128), jnp.float32)],
    )(work)

    nt = nv * nh
    res_steps = nv_res * nh

    out, idx = pl.pallas_call(
        _make_mega_kernel(k_res, pbh, cw, nv_res, nh),
        grid=(npass, nt),
        in_specs=[
            pl.BlockSpec(
                (pbh, cw),
                lambda p, t, R=res_steps, NH=nh, NVR=nv_res:
                    (jnp.where((p == 0) & (t < R), t // NH, NVR - 1),
                     jnp.where((p == 0) & (t < R), t % NH, NH - 1))),
            pl.BlockSpec(
                (pbh, cw),
                lambda p, t, R=res_steps, NH=nh, NVR=nv_res:
                    (jnp.where((p >= 1) & (t >= R), t // NH - NVR, 0),
                     jnp.where((p >= 1) & (t >= R), t % NH, 0))),
            pl.BlockSpec((1, mb), lambda p, t: (0, 0)),
        ],
        out_specs=[
            pl.BlockSpec(
                (pbh, cw),
                lambda p, t, P=npass - 1, NH=nh:
                    (jnp.where(p == P, t // NH, 0),
                     jnp.where(p == P, t % NH, 0))),
            pl.BlockSpec((1, m), lambda p, t: (0, 0)),
        ],
        out_shape=[
            jax.ShapeDtypeStruct((m, n), jnp.float32),
            jax.ShapeDtypeStruct((1, m), jnp.int32),
        ],
        scratch_shapes=[
            pltpu.VMEM((res_steps, pbh, cw), jnp.float32),
            pltpu.VMEM((m, 1), jnp.float32),
            pltpu.VMEM((m, 128), jnp.float32),
            pltpu.VMEM((8, n), jnp.float32),
            pltpu.VMEM((1, n), jnp.float32),
            pltpu.VMEM((m, 1), jnp.int32),
        ],
        compiler_params=pltpu.CompilerParams(
            vmem_limit_bytes=64 * 1024 * 1024,
        ),
    )(work, s_bot, bsum)

    row_ids = jnp.arange(m, dtype=jnp.int32)
    col_ids = idx.reshape(m)
    if num_row >= num_col:
        indices = jnp.stack((col_ids, row_ids), axis=0)
    else:
        indices = jnp.stack((row_ids, col_ids), axis=0)
    values = jnp.ones((m,), dtype=jnp.float32)
    return (out, indices, values)


# speedup vs baseline: 2.0454x; 2.0454x over previous
import jax
import jax.numpy as jnp
from jax.experimental import pallas as pl
import jax.experimental.pallas.tpu as pltpu


def _fold128(t):
    acc = t[:, 0:128]
    for q in range(1, t.shape[1] // 128):
        acc = acc + t[:, q * 128:(q + 1) * 128]
    return acc


def _fold8(t):
    acc = t[0:8, :]
    for q in range(1, t.shape[0] // 8):
        acc = acc + t[q * 8:(q + 1) * 8, :]
    return acc


def _rowsum(t):
    return jnp.sum(_fold128(t), axis=1, keepdims=True)


def _sweep_first_kernel(x_ref, r_ref, c_ref, acc_ref):
    i = pl.program_id(0)
    ni = pl.num_programs(0)
    s = jnp.exp(x_ref[...] * 50.0)
    rp = 1.0 / _rowsum(s)
    r_ref[...] = rp
    part8 = _fold8(s * rp)

    @pl.when(i == 0)
    def _():
        acc_ref[...] = part8

    @pl.when(i != 0)
    def _():
        acc_ref[...] += part8

    @pl.when(i == ni - 1)
    def _():
        c_ref[...] = 1.0 / jnp.sum(acc_ref[...], axis=0, keepdims=True)


def _sweep_kernel(x_ref, cin_ref, r_ref, c_ref, acc_ref):
    i = pl.program_id(0)
    ni = pl.num_programs(0)
    s = jnp.exp(x_ref[...] * 50.0)
    rp = 1.0 / _rowsum(s * cin_ref[...])
    r_ref[...] = rp
    part8 = _fold8(s * rp)

    @pl.when(i == 0)
    def _():
        acc_ref[...] = part8

    @pl.when(i != 0)
    def _():
        acc_ref[...] += part8

    @pl.when(i == ni - 1)
    def _():
        c_ref[...] = 1.0 / jnp.sum(acc_ref[...], axis=0, keepdims=True)


def _output_kernel(x_ref, r_ref, c_ref, out_ref, idx_ref):
    o = jnp.exp(x_ref[...] * 50.0) * r_ref[...] * c_ref[...]
    out_ref[...] = o
    idx_ref[...] = jnp.argmax(o, axis=1).reshape(o.shape[0], 1).astype(
        jnp.int32)


def kernel(sims, batch_size=256):
    del batch_size
    num_row, num_col = sims.shape
    work = sims.T if num_row >= num_col else sims
    m, n = work.shape

    pb = min(256, m)

    grid = (m // pb,)
    x_spec = pl.BlockSpec((pb, n), lambda i: (i, 0))
    r_spec = pl.BlockSpec((pb, 1), lambda i: (i, 0))
    c_spec = pl.BlockSpec((1, n), lambda i: (0, 0))
    vec_shapes = [
        jax.ShapeDtypeStruct((m, 1), jnp.float32),
        jax.ShapeDtypeStruct((1, n), jnp.float32),
    ]
    acc = [pltpu.VMEM((8, n), jnp.float32)]

    r, c = pl.pallas_call(
        _sweep_first_kernel,
        grid=grid,
        in_specs=[x_spec],
        out_specs=[r_spec, c_spec],
        out_shape=vec_shapes,
        scratch_shapes=acc,
    )(work)

    sweep = pl.pallas_call(
        _sweep_kernel,
        grid=grid,
        in_specs=[x_spec, c_spec],
        out_specs=[r_spec, c_spec],
        out_shape=vec_shapes,
        scratch_shapes=acc,
    )
    for _ in range(4):
        r, c = sweep(work, c)

    out, idx = pl.pallas_call(
        _output_kernel,
        grid=grid,
        in_specs=[x_spec, r_spec, c_spec],
        out_specs=[
            x_spec,
            pl.BlockSpec((pb, 1), lambda i: (i, 0)),
        ],
        out_shape=[
            jax.ShapeDtypeStruct((m, n), jnp.float32),
            jax.ShapeDtypeStruct((m, 1), jnp.int32),
        ],
    )(work, r, c)

    row_ids = jnp.arange(m, dtype=jnp.int32)
    col_ids = idx.reshape(m)
    if num_row >= num_col:
        indices = jnp.stack((col_ids, row_ids), axis=0)
    else:
        indices = jnp.stack((row_ids, col_ids), axis=0)
    values = jnp.ones((m,), dtype=jnp.float32)
    return (out, indices, values)


# speedup vs baseline: 2.0643x; 1.0093x over previous
import jax
import jax.numpy as jnp
from jax.experimental import pallas as pl
import jax.experimental.pallas.tpu as pltpu


def _fold128(t):
    acc = t[:, 0:128]
    for q in range(1, t.shape[1] // 128):
        acc = acc + t[:, q * 128:(q + 1) * 128]
    return acc


def _fold8(t):
    acc = t[0:8, :]
    for q in range(1, t.shape[0] // 8):
        acc = acc + t[q * 8:(q + 1) * 8, :]
    return acc


def _rowsum(t):
    return jnp.sum(_fold128(t), axis=1, keepdims=True)


def _sweep_first_kernel(x_ref, r_ref, c_ref, acc_ref):
    i = pl.program_id(0)
    ni = pl.num_programs(0)
    s = jnp.exp(x_ref[...] * 50.0)
    rp = 1.0 / _rowsum(s)
    r_ref[...] = rp
    part8 = _fold8(s * rp)

    @pl.when(i == 0)
    def _():
        acc_ref[...] = part8

    @pl.when(i != 0)
    def _():
        acc_ref[...] += part8

    @pl.when(i == ni - 1)
    def _():
        c_ref[...] = 1.0 / jnp.sum(acc_ref[...], axis=0, keepdims=True)


def _sweep_kernel(x_ref, cin_ref, r_ref, c_ref, acc_ref):
    i = pl.program_id(0)
    ni = pl.num_programs(0)
    s = jnp.exp(x_ref[...] * 50.0)
    rp = 1.0 / _rowsum(s * cin_ref[...])
    r_ref[...] = rp
    part8 = _fold8(s * rp)

    @pl.when(i == 0)
    def _():
        acc_ref[...] = part8

    @pl.when(i != 0)
    def _():
        acc_ref[...] += part8

    @pl.when(i == ni - 1)
    def _():
        c_ref[...] = 1.0 / jnp.sum(acc_ref[...], axis=0, keepdims=True)


def _output_kernel(x_ref, r_ref, c_ref, out_ref, idx_ref):
    o = jnp.exp(x_ref[...] * 50.0) * r_ref[...] * c_ref[...]
    out_ref[...] = o
    idx_ref[...] = jnp.argmax(o, axis=1).reshape(o.shape[0], 1).astype(
        jnp.int32)


def kernel(sims, batch_size=256):
    del batch_size
    num_row, num_col = sims.shape
    work = sims.T if num_row >= num_col else sims
    m, n = work.shape

    pb = min(512, m)
    po = min(256, m)

    grid = (m // pb,)
    x_spec = pl.BlockSpec((pb, n), lambda i: (i, 0))
    r_spec = pl.BlockSpec((pb, 1), lambda i: (i, 0))
    c_spec = pl.BlockSpec((1, n), lambda i: (0, 0))
    vec_shapes = [
        jax.ShapeDtypeStruct((m, 1), jnp.float32),
        jax.ShapeDtypeStruct((1, n), jnp.float32),
    ]
    acc = [pltpu.VMEM((8, n), jnp.float32)]

    r, c = pl.pallas_call(
        _sweep_first_kernel,
        grid=grid,
        in_specs=[x_spec],
        out_specs=[r_spec, c_spec],
        out_shape=vec_shapes,
        scratch_shapes=acc,
    )(work)

    sweep = pl.pallas_call(
        _sweep_kernel,
        grid=grid,
        in_specs=[x_spec, c_spec],
        out_specs=[r_spec, c_spec],
        out_shape=vec_shapes,
        scratch_shapes=acc,
    )
    for _ in range(4):
        r, c = sweep(work, c)

    out, idx = pl.pallas_call(
        _output_kernel,
        grid=(m // po,),
        in_specs=[
            pl.BlockSpec((po, n), lambda i: (i, 0)),
            pl.BlockSpec((po, 1), lambda i: (i, 0)),
            pl.BlockSpec((1, n), lambda i: (0, 0)),
        ],
        out_specs=[
            pl.BlockSpec((po, n), lambda i: (i, 0)),
            pl.BlockSpec((po, 1), lambda i: (i, 0)),
        ],
        out_shape=[
            jax.ShapeDtypeStruct((m, n), jnp.float32),
            jax.ShapeDtypeStruct((m, 1), jnp.int32),
        ],
    )(work, r, c)

    row_ids = jnp.arange(m, dtype=jnp.int32)
    col_ids = idx.reshape(m)
    if num_row >= num_col:
        indices = jnp.stack((col_ids, row_ids), axis=0)
    else:
        indices = jnp.stack((row_ids, col_ids), axis=0)
    values = jnp.ones((m,), dtype=jnp.float32)
    return (out, indices, values)
